# Initial kernel scaffold; baseline (speedup 1.0000x reference)
import jax
import jax.numpy as jnp
from jax.experimental import pallas as pl


def kernel(loc_data, conf_data, priors, targets):
    raise NotImplementedError("write your pallas kernel here")


# speedup vs baseline: 27.6590x; 27.6590x over previous
import jax
import jax.numpy as jnp
from jax import lax
from jax.experimental import pallas as pl
from jax.experimental.pallas import tpu as pltpu

_B = 32
_P = 8732
_C = 21
_O = 20
_NEG_POS = 3
_THRESH = 0.5
_V0 = 0.1
_V1 = 0.2


def _body(loc_ref, conf_ref, pri_ref, tgt_ref, out_ref, nl_ref, meta_ref):
    b = pl.program_id(0)

    @pl.when(b < _B)
    def _per_sample():
        tgt = tgt_ref[0]
        tx1 = tgt[:, 0:1]
        ty1 = tgt[:, 1:2]
        tx2 = tgt[:, 2:3]
        ty2 = tgt[:, 3:4]
        lbl = tgt[:, 4:5]

        pri = pri_ref[...]
        pcx = pri[0:1, :]
        pcy = pri[1:2, :]
        pw = pri[2:3, :]
        ph = pri[3:4, :]
        px1 = pcx - pw * 0.5
        py1 = pcy - ph * 0.5
        px2 = pcx + pw * 0.5
        py2 = pcy + ph * 0.5

        iw = jnp.maximum(jnp.minimum(tx2, px2) - jnp.maximum(tx1, px1), 0.0)
        ih = jnp.maximum(jnp.minimum(ty2, py2) - jnp.maximum(ty1, py1), 0.0)
        inter = iw * ih
        area_t = (tx2 - tx1) * (ty2 - ty1)
        area_p = (px2 - px1) * (py2 - py1)
        ov = inter / (area_t + area_p - inter)

        lane = lax.broadcasted_iota(jnp.int32, (_O, _P), 1)
        sub = lax.broadcasted_iota(jnp.int32, (_O, _P), 0)

        bpo = jnp.max(ov, axis=1, keepdims=True)
        bpi = jnp.min(jnp.where(ov == bpo, lane, _P), axis=1, keepdims=True)
        bto = jnp.max(ov, axis=0, keepdims=True)
        bti = jnp.min(jnp.where(ov == bto, sub, _O), axis=0, keepdims=True)

        hit = lane == bpi
        mo = jnp.max(jnp.where(hit, sub, -1), axis=0, keepdims=True)
        forced = mo >= 0
        bto = jnp.where(forced, 2.0, bto)
        bti = jnp.where(forced, mo, bti)

        oh = sub == bti

        def gath(col):
            return jnp.sum(jnp.where(oh, col, 0.0), axis=0, keepdims=True)

        mx1 = gath(tx1)
        my1 = gath(ty1)
        mx2 = gath(tx2)
        my2 = gath(ty2)
        mlb = gath(lbl)

        conf_t = jnp.where(bto < _THRESH, 0.0, mlb)
        pos = conf_t > 0.0

        g_cx = ((mx1 + mx2) * 0.5 - pcx) / (_V0 * pw)
        g_cy = ((my1 + my2) * 0.5 - pcy) / (_V0 * ph)
        g_w = jnp.log((mx2 - mx1) / pw) / _V1
        g_h = jnp.log((my2 - my1) / ph) / _V1

        loc = loc_ref[0]

        def sl1(x):
            ax = jnp.abs(x)
            return jnp.where(ax < 1.0, 0.5 * x * x, ax - 0.5)

        l_elem = (sl1(loc[0:1, :] - g_cx) + sl1(loc[1:2, :] - g_cy)
                  + sl1(loc[2:3, :] - g_w) + sl1(loc[3:4, :] - g_h))
        loss_l = jnp.sum(jnp.where(pos, l_elem, 0.0))

        cf = conf_ref[0]
        m = jnp.max(cf, axis=0, keepdims=True)
        lse = jnp.log(jnp.sum(jnp.exp(cf - m), axis=0, keepdims=True)) + m
        csub = lax.broadcasted_iota(jnp.int32, (_C, _P), 0)
        selhot = csub == conf_t.astype(jnp.int32)
        xsel = jnp.sum(jnp.where(selhot, cf, 0.0), axis=0, keepdims=True)
        ce = lse - xsel

        nl_ref[pl.ds(b, 1), :] = jnp.where(pos, 0.0, ce)
        npos = jnp.sum(jnp.where(pos, 1.0, 0.0))
        cepos = jnp.sum(jnp.where(pos, ce, 0.0))
        lane128 = lax.broadcasted_iota(jnp.int32, (1, 128), 1)
        meta = jnp.where(lane128 == 0, npos,
                         jnp.where(lane128 == 1, loss_l,
                                   jnp.where(lane128 == 2, cepos, 0.0)))
        meta_ref[pl.ds(b, 1), :] = meta

    @pl.when(b == _B)
    def _finalize():
        meta = meta_ref[...]
        npos = meta[:, 0:1]
        k = jnp.minimum(npos.astype(jnp.int32) * _NEG_POS, _P - 1)
        v = nl_ref[...]
        vi = lax.bitcast_convert_type(v, jnp.int32)

        hi0 = jnp.max(vi, axis=1, keepdims=True)
        lo0 = jnp.full_like(hi0, -1)

        def step(_, carry):
            lo, hi = carry
            mid = lo + (hi - lo) // 2
            cnt = jnp.sum(jnp.where(vi > mid, 1, 0), axis=1, keepdims=True)
            pred = cnt >= k
            return jnp.where(pred, mid, lo), jnp.where(pred, hi, mid)

        _, t_i = lax.fori_loop(0, 31, step, (lo0, hi0))
        t_f = lax.bitcast_convert_type(t_i, jnp.float32)
        gt = vi > t_i
        cnt_t = jnp.sum(jnp.where(gt, 1, 0), axis=1, keepdims=True)
        sum_gt = jnp.sum(jnp.where(gt, v, 0.0), axis=1, keepdims=True)
        topk = sum_gt + (k - cnt_t).astype(jnp.float32) * t_f

        n = jnp.sum(meta[:, 0:1])
        ll = jnp.sum(meta[:, 1:2])
        lc = jnp.sum(meta[:, 2:3]) + jnp.sum(topk)
        subi = lax.broadcasted_iota(jnp.int32, (8, 128), 0)
        out_ref[...] = jnp.where(subi == 0, ll / n,
                                 jnp.where(subi == 1, lc / n, 0.0))


def kernel(loc_data, conf_data, priors, targets):
    loc_t = loc_data.transpose(0, 2, 1)
    conf_t = conf_data.transpose(0, 2, 1)
    pri_t = priors.T

    out = pl.pallas_call(
        _body,
        grid=(_B + 1,),
        in_specs=[
            pl.BlockSpec((1, 4, _P), lambda b: (jnp.minimum(b, _B - 1), 0, 0)),
            pl.BlockSpec((1, _C, _P), lambda b: (jnp.minimum(b, _B - 1), 0, 0)),
            pl.BlockSpec((4, _P), lambda b: (0, 0)),
            pl.BlockSpec((1, _O, 5), lambda b: (jnp.minimum(b, _B - 1), 0, 0)),
        ],
        out_specs=pl.BlockSpec((8, 128), lambda b: (0, 0)),
        out_shape=jax.ShapeDtypeStruct((8, 128), jnp.float32),
        scratch_shapes=[
            pltpu.VMEM((_B, _P), jnp.float32),
            pltpu.VMEM((_B, 128), jnp.float32),
        ],
    )(loc_t, conf_t, pri_t, targets)

    return out[0, 0], out[1, 0]


# speedup vs baseline: 34.8011x; 1.2582x over previous
import jax
import jax.numpy as jnp
from jax import lax
from jax.experimental import pallas as pl
from jax.experimental.pallas import tpu as pltpu

_B = 32
_P = 8732
_C = 21
_O = 20
_NEG_POS = 3
_THRESH = 0.5
_V0 = 0.1
_V1 = 0.2


def _body(loc_ref, conf_ref, pri_ref, tgt_ref, tgt5_ref, out_ref, nl_ref,
          meta_ref):
    b = pl.program_id(0)

    @pl.when(b < _B)
    def _per_sample():
        tgt = tgt_ref[0]
        tx1 = tgt[:, 0:1]
        ty1 = tgt[:, 1:2]
        tx2 = tgt[:, 2:3]
        ty2 = tgt[:, 3:4]

        pri = pri_ref[...]
        pcx = pri[0:1, :]
        pcy = pri[1:2, :]
        pw = pri[2:3, :]
        ph = pri[3:4, :]
        px1 = pcx - pw * 0.5
        py1 = pcy - ph * 0.5
        px2 = pcx + pw * 0.5
        py2 = pcy + ph * 0.5

        iw = jnp.maximum(jnp.minimum(tx2, px2) - jnp.maximum(tx1, px1), 0.0)
        ih = jnp.maximum(jnp.minimum(ty2, py2) - jnp.maximum(ty1, py1), 0.0)
        inter = iw * ih
        area_t = (tx2 - tx1) * (ty2 - ty1)
        area_p = (px2 - px1) * (py2 - py1)
        ov = inter / (area_t + area_p - inter)

        lane = lax.broadcasted_iota(jnp.int32, (_O, _P), 1)
        sub = lax.broadcasted_iota(jnp.int32, (_O, _P), 0)

        bpo = jnp.max(ov, axis=1, keepdims=True)
        bpi = jnp.min(jnp.where(ov == bpo, lane, _P), axis=1, keepdims=True)
        bto = jnp.max(ov, axis=0, keepdims=True)
        bti = jnp.min(jnp.where(ov == bto, sub, _O), axis=0, keepdims=True)

        hit = lane == bpi
        mo = jnp.max(jnp.where(hit, sub, -1), axis=0, keepdims=True)
        forced = mo >= 0
        bto = jnp.where(forced, 2.0, bto)
        bti = jnp.where(forced, mo, bti)

        oh = jnp.where(sub == bti, 1.0, 0.0)
        mat = jax.lax.dot_general(
            tgt5_ref[0], oh, (((1,), (0,)), ((), ())),
            preferred_element_type=jnp.float32)
        mx1 = mat[0:1, :]
        my1 = mat[1:2, :]
        mx2 = mat[2:3, :]
        my2 = mat[3:4, :]
        mlb = mat[4:5, :]

        conf_t = jnp.where(bto < _THRESH, 0.0, mlb)
        pos = conf_t > 0.0

        g_cx = ((mx1 + mx2) * 0.5 - pcx) / (_V0 * pw)
        g_cy = ((my1 + my2) * 0.5 - pcy) / (_V0 * ph)
        g_w = jnp.log((mx2 - mx1) / pw) / _V1
        g_h = jnp.log((my2 - my1) / ph) / _V1

        loc = loc_ref[0]

        def sl1(x):
            ax = jnp.abs(x)
            return jnp.where(ax < 1.0, 0.5 * x * x, ax - 0.5)

        l_elem = (sl1(loc[0:1, :] - g_cx) + sl1(loc[1:2, :] - g_cy)
                  + sl1(loc[2:3, :] - g_w) + sl1(loc[3:4, :] - g_h))
        loss_l = jnp.sum(jnp.where(pos, l_elem, 0.0))

        cf = conf_ref[0]
        m = jnp.max(cf, axis=0, keepdims=True)
        ex = jnp.exp(cf - m)
        csub = lax.broadcasted_iota(jnp.int32, (_C, _P), 0)
        sel = jnp.where(csub == conf_t.astype(jnp.int32), cf, 0.0)
        ones_c = jnp.ones((1, _C), dtype=jnp.float32)
        ssum = jax.lax.dot_general(
            ones_c, ex, (((1,), (0,)), ((), ())),
            preferred_element_type=jnp.float32)
        xsel = jax.lax.dot_general(
            ones_c, sel, (((1,), (0,)), ((), ())),
            preferred_element_type=jnp.float32)
        ce = jnp.log(ssum) + m - xsel

        nl_ref[pl.ds(b, 1), :] = jnp.where(pos, 0.0, ce)
        npos = jnp.sum(jnp.where(pos, 1.0, 0.0))
        cepos = jnp.sum(jnp.where(pos, ce, 0.0))
        lane128 = lax.broadcasted_iota(jnp.int32, (1, 128), 1)
        meta = jnp.where(lane128 == 0, npos,
                         jnp.where(lane128 == 1, loss_l,
                                   jnp.where(lane128 == 2, cepos, 0.0)))
        meta_ref[pl.ds(b, 1), :] = meta

    @pl.when(b == _B)
    def _finalize():
        meta = meta_ref[...]
        npos = meta[:, 0:1]
        kf = jnp.minimum(npos * _NEG_POS, float(_P - 1))
        v = nl_ref[...]
        vi = lax.bitcast_convert_type(v, jnp.int32)
        ones_p = jnp.ones((_P, 1), dtype=jnp.float32)

        def row_sum(x):
            return jax.lax.dot_general(
                x, ones_p, (((1,), (0,)), ((), ())),
                preferred_element_type=jnp.float32)

        hi0 = jnp.max(vi, axis=1, keepdims=True)
        lo0 = jnp.full_like(hi0, -1)

        def step(_, carry):
            lo, hi = carry
            mid = lo + (hi - lo) // 2
            cnt = row_sum(jnp.where(vi > mid, 1.0, 0.0))
            pred = cnt >= kf
            return jnp.where(pred, mid, lo), jnp.where(pred, hi, mid)

        _, t_i = lax.fori_loop(0, 31, step, (lo0, hi0))
        t_f = lax.bitcast_convert_type(t_i, jnp.float32)
        gt = vi > t_i
        cnt_t = row_sum(jnp.where(gt, 1.0, 0.0))
        sum_gt = row_sum(jnp.where(gt, v, 0.0))
        topk = sum_gt + (kf - cnt_t) * t_f

        n = jnp.sum(meta[:, 0:1])
        ll = jnp.sum(meta[:, 1:2])
        lc = jnp.sum(meta[:, 2:3]) + jnp.sum(topk)
        subi = lax.broadcasted_iota(jnp.int32, (8, 128), 0)
        out_ref[...] = jnp.where(subi == 0, ll / n,
                                 jnp.where(subi == 1, lc / n, 0.0))


def kernel(loc_data, conf_data, priors, targets):
    loc_t = loc_data.transpose(0, 2, 1)
    conf_t = conf_data.transpose(0, 2, 1)
    pri_t = priors.T

    out = pl.pallas_call(
        _body,
        grid=(_B + 1,),
        in_specs=[
            pl.BlockSpec((1, 4, _P), lambda b: (jnp.minimum(b, _B - 1), 0, 0)),
            pl.BlockSpec((1, _C, _P), lambda b: (jnp.minimum(b, _B - 1), 0, 0)),
            pl.BlockSpec((4, _P), lambda b: (0, 0)),
            pl.BlockSpec((1, _O, 5), lambda b: (jnp.minimum(b, _B - 1), 0, 0)),
            pl.BlockSpec((1, 5, _O), lambda b: (jnp.minimum(b, _B - 1), 0, 0)),
        ],
        out_specs=pl.BlockSpec((8, 128), lambda b: (0, 0)),
        out_shape=jax.ShapeDtypeStruct((8, 128), jnp.float32),
        scratch_shapes=[
            pltpu.VMEM((_B, _P), jnp.float32),
            pltpu.VMEM((_B, 128), jnp.float32),
        ],
    )(loc_t, conf_t, pri_t, targets, targets.transpose(0, 2, 1))

    return out[0, 0], out[1, 0]


# speedup vs baseline: 37.6127x; 1.0808x over previous
import jax
import jax.numpy as jnp
from jax import lax
from jax.experimental import pallas as pl
from jax.experimental.pallas import tpu as pltpu

_B = 32
_P = 8732
_C = 21
_O = 20
_NEG_POS = 3
_THRESH = 0.5
_V0 = 0.1
_V1 = 0.2


def _body(loc_ref, conf_ref, pri_ref, tgt_ref, tgt5_ref, out_ref, nl_ref,
          meta_ref):
    b = pl.program_id(0)

    @pl.when(b < _B)
    def _per_sample():
        tgt = tgt_ref[0]
        tx1 = tgt[:, 0:1]
        ty1 = tgt[:, 1:2]
        tx2 = tgt[:, 2:3]
        ty2 = tgt[:, 3:4]

        pri = pri_ref[...]
        pcx = pri[0:1, :]
        pcy = pri[1:2, :]
        pw = pri[2:3, :]
        ph = pri[3:4, :]
        px1 = pcx - pw * 0.5
        py1 = pcy - ph * 0.5
        px2 = pcx + pw * 0.5
        py2 = pcy + ph * 0.5

        iw = jnp.maximum(jnp.minimum(tx2, px2) - jnp.maximum(tx1, px1), 0.0)
        ih = jnp.maximum(jnp.minimum(ty2, py2) - jnp.maximum(ty1, py1), 0.0)
        inter = iw * ih
        area_t = (tx2 - tx1) * (ty2 - ty1)
        area_p = (px2 - px1) * (py2 - py1)
        ov = inter / (area_t + area_p - inter)

        lane = lax.broadcasted_iota(jnp.int32, (_O, _P), 1)
        sub = lax.broadcasted_iota(jnp.int32, (_O, _P), 0)

        bpo = jnp.max(ov, axis=1, keepdims=True)
        bpi = jnp.min(jnp.where(ov == bpo, lane, _P), axis=1, keepdims=True)
        bto = jnp.max(ov, axis=0, keepdims=True)
        bti = jnp.min(jnp.where(ov == bto, sub, _O), axis=0, keepdims=True)

        hit = lane == bpi
        mo = jnp.max(jnp.where(hit, sub, -1), axis=0, keepdims=True)
        forced = mo >= 0
        bto = jnp.where(forced, 2.0, bto)
        bti = jnp.where(forced, mo, bti)

        oh = jnp.where(sub == bti, 1.0, 0.0)
        mat = jax.lax.dot_general(
            tgt5_ref[0], oh, (((1,), (0,)), ((), ())),
            preferred_element_type=jnp.float32)
        m_lo = mat[0:2, :]
        m_hi = mat[2:4, :]
        mlb = mat[4:5, :]

        conf_t = jnp.where(bto < _THRESH, 0.0, mlb)
        pos = conf_t > 0.0

        pc = pri[0:2, :]
        inv_pwh = 1.0 / pri[2:4, :]
        g_cxy = ((m_lo + m_hi) * 0.5 - pc) * ((1.0 / _V0) * inv_pwh)
        g_wh = jnp.log((m_hi - m_lo) * inv_pwh) * (1.0 / _V1)
        g = jnp.concatenate([g_cxy, g_wh], axis=0)

        diff = loc_ref[0] - g
        ax = jnp.abs(diff)
        l_elem = jnp.sum(
            jnp.where(ax < 1.0, 0.5 * diff * diff, ax - 0.5),
            axis=0, keepdims=True)
        loss_l = jnp.sum(jnp.where(pos, l_elem, 0.0))

        cf = conf_ref[0]
        m = jnp.max(cf, axis=0, keepdims=True)
        ex = jnp.exp(cf - m)
        csub = lax.broadcasted_iota(jnp.int32, (_C, _P), 0)
        sel = jnp.where(csub == conf_t.astype(jnp.int32), cf, 0.0)
        ones_c = jnp.ones((1, _C), dtype=jnp.float32)
        ssum = jax.lax.dot_general(
            ones_c, ex, (((1,), (0,)), ((), ())),
            preferred_element_type=jnp.float32)
        xsel = jax.lax.dot_general(
            ones_c, sel, (((1,), (0,)), ((), ())),
            preferred_element_type=jnp.float32)
        ce = jnp.maximum(jnp.log(ssum) + m - xsel, 0.0)

        nl_ref[pl.ds(b, 1), :] = jnp.where(pos, 0.0, ce)
        npos = jnp.sum(jnp.where(pos, 1.0, 0.0))
        cepos = jnp.sum(jnp.where(pos, ce, 0.0))
        lane128 = lax.broadcasted_iota(jnp.int32, (1, 128), 1)
        meta = jnp.where(lane128 == 0, npos,
                         jnp.where(lane128 == 1, loss_l,
                                   jnp.where(lane128 == 2, cepos, 0.0)))
        meta_ref[pl.ds(b, 1), :] = meta

    @pl.when(b == _B)
    def _finalize():
        meta = meta_ref[...]
        npos = meta[:, 0:1]
        k = jnp.minimum(npos.astype(jnp.int32) * _NEG_POS, _P - 1)
        v = nl_ref[...]
        vi = lax.bitcast_convert_type(v, jnp.int32)

        hi0 = jnp.max(vi, axis=1, keepdims=True)
        lo0 = jnp.full_like(hi0, -1)

        def step(_, carry):
            lo, hi = carry
            mid = lo + (hi - lo) // 2
            cnt = jnp.sum(vi > mid, axis=1, keepdims=True, dtype=jnp.int32)
            pred = cnt >= k
            return jnp.where(pred, mid, lo), jnp.where(pred, hi, mid)

        _, t_i = lax.fori_loop(0, 31, step, (lo0, hi0))
        t_f = lax.bitcast_convert_type(t_i, jnp.float32)
        gt = vi > t_i
        cnt_t = jnp.sum(gt, axis=1, keepdims=True, dtype=jnp.int32)
        sum_gt = jnp.sum(jnp.where(gt, v, 0.0), axis=1, keepdims=True)
        topk = sum_gt + (k - cnt_t).astype(jnp.float32) * t_f

        n = jnp.sum(meta[:, 0:1])
        ll = jnp.sum(meta[:, 1:2])
        lc = jnp.sum(meta[:, 2:3]) + jnp.sum(topk)
        subi = lax.broadcasted_iota(jnp.int32, (8, 128), 0)
        out_ref[...] = jnp.where(subi == 0, ll / n,
                                 jnp.where(subi == 1, lc / n, 0.0))


def kernel(loc_data, conf_data, priors, targets):
    loc_t = loc_data.transpose(0, 2, 1)
    conf_t = conf_data.transpose(0, 2, 1)
    pri_t = priors.T

    out = pl.pallas_call(
        _body,
        grid=(_B + 1,),
        in_specs=[
            pl.BlockSpec((1, 4, _P), lambda b: (jnp.minimum(b, _B - 1), 0, 0)),
            pl.BlockSpec((1, _C, _P), lambda b: (jnp.minimum(b, _B - 1), 0, 0)),
            pl.BlockSpec((4, _P), lambda b: (0, 0)),
            pl.BlockSpec((1, _O, 5), lambda b: (jnp.minimum(b, _B - 1), 0, 0)),
            pl.BlockSpec((1, 5, _O), lambda b: (jnp.minimum(b, _B - 1), 0, 0)),
        ],
        out_specs=pl.BlockSpec((8, 128), lambda b: (0, 0)),
        out_shape=jax.ShapeDtypeStruct((8, 128), jnp.float32),
        scratch_shapes=[
            pltpu.VMEM((_B, _P), jnp.float32),
            pltpu.VMEM((_B, 128), jnp.float32),
        ],
    )(loc_t, conf_t, pri_t, targets, targets.transpose(0, 2, 1))

    return out[0, 0], out[1, 0]


# speedup vs baseline: 39.8581x; 1.0597x over previous
import jax
import jax.numpy as jnp
from jax import lax
from jax.experimental import pallas as pl
from jax.experimental.pallas import tpu as pltpu

_B = 32
_P = 8732
_C = 21
_O = 20
_NEG_POS = 3
_THRESH = 0.5
_V0 = 0.1
_V1 = 0.2


def _match_body(loc_ref, pri_ref, tgt_ref, tgt5_ref, ct_ref, meta_ref):
    tgt = tgt_ref[0]
    tx1 = tgt[:, 0:1]
    ty1 = tgt[:, 1:2]
    tx2 = tgt[:, 2:3]
    ty2 = tgt[:, 3:4]

    pri = pri_ref[...]
    pcx = pri[0:1, :]
    pcy = pri[1:2, :]
    pw = pri[2:3, :]
    ph = pri[3:4, :]
    px1 = pcx - pw * 0.5
    py1 = pcy - ph * 0.5
    px2 = pcx + pw * 0.5
    py2 = pcy + ph * 0.5

    iw = jnp.maximum(jnp.minimum(tx2, px2) - jnp.maximum(tx1, px1), 0.0)
    ih = jnp.maximum(jnp.minimum(ty2, py2) - jnp.maximum(ty1, py1), 0.0)
    inter = iw * ih
    area_t = (tx2 - tx1) * (ty2 - ty1)
    area_p = (px2 - px1) * (py2 - py1)
    ov = inter / (area_t + area_p - inter)

    lane = lax.broadcasted_iota(jnp.int32, (_O, _P), 1)
    sub = lax.broadcasted_iota(jnp.int32, (_O, _P), 0)

    bpo = jnp.max(ov, axis=1, keepdims=True)
    bpi = jnp.min(jnp.where(ov == bpo, lane, _P), axis=1, keepdims=True)
    bto = jnp.max(ov, axis=0, keepdims=True)
    bti = jnp.min(jnp.where(ov == bto, sub, _O), axis=0, keepdims=True)

    hit = lane == bpi
    mo = jnp.max(jnp.where(hit, sub, -1), axis=0, keepdims=True)
    forced = mo >= 0
    bto = jnp.where(forced, 2.0, bto)
    bti = jnp.where(forced, mo, bti)

    oh = jnp.where(sub == bti, 1.0, 0.0)
    mat = jax.lax.dot_general(
        tgt5_ref[0], oh, (((1,), (0,)), ((), ())),
        preferred_element_type=jnp.float32)
    m_lo = mat[0:2, :]
    m_hi = mat[2:4, :]
    mlb = mat[4:5, :]

    conf_t = jnp.where(bto < _THRESH, 0.0, mlb)
    pos = conf_t > 0.0

    pc = pri[0:2, :]
    inv_pwh = 1.0 / pri[2:4, :]
    g_cxy = ((m_lo + m_hi) * 0.5 - pc) * ((1.0 / _V0) * inv_pwh)
    g_wh = jnp.log((m_hi - m_lo) * inv_pwh) * (1.0 / _V1)
    g = jnp.concatenate([g_cxy, g_wh], axis=0)

    diff = loc_ref[0] - g
    ax = jnp.abs(diff)
    l_elem = jnp.sum(
        jnp.where(ax < 1.0, 0.5 * diff * diff, ax - 0.5),
        axis=0, keepdims=True)
    loss_l = jnp.sum(jnp.where(pos, l_elem, 0.0))
    npos = jnp.sum(jnp.where(pos, 1.0, 0.0))

    ct_ref[0] = conf_t
    lane128 = lax.broadcasted_iota(jnp.int32, (1, 128), 1)
    meta_ref[0] = jnp.where(lane128 == 0, npos,
                            jnp.where(lane128 == 1, loss_l, 0.0))


def _loss_body(conf_ref, ct_ref, meta_ref, out_ref, nl_ref, cp_ref):
    b = pl.program_id(0)

    @pl.when(b < _B)
    def _per_sample():
        conf_t = ct_ref[0]
        pos = conf_t > 0.0

        cf = conf_ref[0]
        m = jnp.max(cf, axis=0, keepdims=True)
        ex = jnp.exp(cf - m)
        csub = lax.broadcasted_iota(jnp.int32, (_C, _P), 0)
        sel = jnp.where(csub == conf_t.astype(jnp.int32), cf, 0.0)
        ones_c = jnp.ones((1, _C), dtype=jnp.float32)
        ssum = jax.lax.dot_general(
            ones_c, ex, (((1,), (0,)), ((), ())),
            preferred_element_type=jnp.float32)
        xsel = jax.lax.dot_general(
            ones_c, sel, (((1,), (0,)), ((), ())),
            preferred_element_type=jnp.float32)
        ce = jnp.maximum(jnp.log(ssum) + m - xsel, 0.0)

        nl_ref[pl.ds(b, 1), :] = jnp.where(pos, 0.0, ce)
        cepos = jnp.sum(jnp.where(pos, ce, 0.0))
        lane128 = lax.broadcasted_iota(jnp.int32, (1, 128), 1)
        cp_ref[pl.ds(b, 1), :] = jnp.where(lane128 == 0, cepos, 0.0)

    @pl.when(b == _B)
    def _finalize():
        meta = meta_ref[:, 0, :]
        npos = meta[:, 0:1]
        k = jnp.minimum(npos.astype(jnp.int32) * _NEG_POS, _P - 1)
        v = nl_ref[...]
        vi = lax.bitcast_convert_type(v, jnp.int32)

        hi0 = jnp.max(vi, axis=1, keepdims=True)
        lo0 = jnp.full_like(hi0, -1)

        def step(_, carry):
            lo, hi = carry
            mid = lo + (hi - lo) // 2
            cnt = jnp.sum(vi > mid, axis=1, keepdims=True, dtype=jnp.int32)
            pred = cnt >= k
            return jnp.where(pred, mid, lo), jnp.where(pred, hi, mid)

        _, t_i = lax.fori_loop(0, 31, step, (lo0, hi0))
        t_f = lax.bitcast_convert_type(t_i, jnp.float32)
        gt = vi > t_i
        cnt_t = jnp.sum(gt, axis=1, keepdims=True, dtype=jnp.int32)
        sum_gt = jnp.sum(jnp.where(gt, v, 0.0), axis=1, keepdims=True)
        topk = sum_gt + (k - cnt_t).astype(jnp.float32) * t_f

        n = jnp.sum(npos)
        ll = jnp.sum(meta[:, 1:2])
        lc = jnp.sum(cp_ref[:, 0:1]) + jnp.sum(topk)
        subi = lax.broadcasted_iota(jnp.int32, (8, 128), 0)
        out_ref[...] = jnp.where(subi == 0, ll / n,
                                 jnp.where(subi == 1, lc / n, 0.0))


def kernel(loc_data, conf_data, priors, targets):
    loc_t = loc_data.transpose(0, 2, 1)
    conf_t = conf_data.transpose(0, 2, 1)
    pri_t = priors.T

    ct, meta = pl.pallas_call(
        _match_body,
        grid=(_B,),
        in_specs=[
            pl.BlockSpec((1, 4, _P), lambda b: (b, 0, 0)),
            pl.BlockSpec((4, _P), lambda b: (0, 0)),
            pl.BlockSpec((1, _O, 5), lambda b: (b, 0, 0)),
            pl.BlockSpec((1, 5, _O), lambda b: (b, 0, 0)),
        ],
        out_specs=[
            pl.BlockSpec((1, 1, _P), lambda b: (b, 0, 0)),
            pl.BlockSpec((1, 1, 128), lambda b: (b, 0, 0)),
        ],
        out_shape=[
            jax.ShapeDtypeStruct((_B, 1, _P), jnp.float32),
            jax.ShapeDtypeStruct((_B, 1, 128), jnp.float32),
        ],
    )(loc_t, pri_t, targets, targets.transpose(0, 2, 1))

    out = pl.pallas_call(
        _loss_body,
        grid=(_B + 1,),
        in_specs=[
            pl.BlockSpec((1, _C, _P), lambda b: (jnp.minimum(b, _B - 1), 0, 0)),
            pl.BlockSpec((1, 1, _P), lambda b: (jnp.minimum(b, _B - 1), 0, 0)),
            pl.BlockSpec((_B, 1, 128), lambda b: (0, 0, 0)),
        ],
        out_specs=pl.BlockSpec((8, 128), lambda b: (0, 0)),
        out_shape=jax.ShapeDtypeStruct((8, 128), jnp.float32),
        scratch_shapes=[
            pltpu.VMEM((_B, _P), jnp.float32),
            pltpu.VMEM((_B, 128), jnp.float32),
        ],
    )(conf_t, ct, meta)

    return out[0, 0], out[1, 0]


# speedup vs baseline: 42.3783x; 1.0632x over previous
import jax
import jax.numpy as jnp
from jax import lax
from jax.experimental import pallas as pl
from jax.experimental.pallas import tpu as pltpu

_B = 32
_P = 8732
_C = 21
_O = 20
_NEG_POS = 3
_THRESH = 0.5
_V0 = 0.1
_V1 = 0.2


def _match_body(loc_ref, pri_ref, tgt_ref, tgt5_ref, ct_ref, meta_ref):
    tgt = tgt_ref[0]
    tx1 = tgt[:, 0:1]
    ty1 = tgt[:, 1:2]
    tx2 = tgt[:, 2:3]
    ty2 = tgt[:, 3:4]

    pri = pri_ref[...]
    pcx = pri[0:1, :]
    pcy = pri[1:2, :]
    pw = pri[2:3, :]
    ph = pri[3:4, :]
    px1 = pcx - pw * 0.5
    py1 = pcy - ph * 0.5
    px2 = pcx + pw * 0.5
    py2 = pcy + ph * 0.5

    iw = jnp.maximum(jnp.minimum(tx2, px2) - jnp.maximum(tx1, px1), 0.0)
    ih = jnp.maximum(jnp.minimum(ty2, py2) - jnp.maximum(ty1, py1), 0.0)
    inter = iw * ih
    area_t = (tx2 - tx1) * (ty2 - ty1)
    area_p = (px2 - px1) * (py2 - py1)
    ov = inter / (area_t + area_p - inter)

    sub = lax.broadcasted_iota(jnp.int32, (_O, _P), 0)

    bpo = jnp.max(ov, axis=1, keepdims=True)
    bto = jnp.max(ov, axis=0, keepdims=True)
    hitm = jnp.logical_and(ov == bpo, bpo > 0.0)
    mo = jnp.max(jnp.where(hitm, sub, -1), axis=0, keepdims=True)
    forced = mo >= 0
    ohm = jnp.where(ov == bto, 1.0, 0.0)
    bto = jnp.where(forced, 2.0, bto)

    oh = jnp.where(forced, jnp.where(sub == mo, 1.0, 0.0), ohm)
    mat = jax.lax.dot_general(
        tgt5_ref[0], oh, (((1,), (0,)), ((), ())),
        preferred_element_type=jnp.float32)
    m_lo = mat[0:2, :]
    m_hi = mat[2:4, :]
    mlb = mat[4:5, :]

    conf_t = jnp.where(bto < _THRESH, 0.0, mlb)
    pos = conf_t > 0.0

    pc = pri[0:2, :]
    inv_pwh = 1.0 / pri[2:4, :]
    g_cxy = ((m_lo + m_hi) * 0.5 - pc) * ((1.0 / _V0) * inv_pwh)
    g_wh = jnp.log((m_hi - m_lo) * inv_pwh) * (1.0 / _V1)
    g = jnp.concatenate([g_cxy, g_wh], axis=0)

    diff = loc_ref[0] - g
    ax = jnp.abs(diff)
    l_elem = jnp.sum(
        jnp.where(ax < 1.0, 0.5 * diff * diff, ax - 0.5),
        axis=0, keepdims=True)
    loss_l = jnp.sum(jnp.where(pos, l_elem, 0.0))
    npos = jnp.sum(jnp.where(pos, 1.0, 0.0))

    ct_ref[0] = conf_t
    lane128 = lax.broadcasted_iota(jnp.int32, (1, 128), 1)
    meta_ref[0] = jnp.where(lane128 == 0, npos,
                            jnp.where(lane128 == 1, loss_l, 0.0))


def _loss_body(conf_ref, ct_ref, meta_ref, out_ref, nl_ref, cp_ref):
    b = pl.program_id(0)

    @pl.when(b < _B)
    def _per_sample():
        conf_t = ct_ref[0]
        pos = conf_t > 0.0

        cf = conf_ref[0]
        m = jnp.max(cf, axis=0, keepdims=True)
        ex = jnp.exp(cf - m)
        csub = lax.broadcasted_iota(jnp.int32, (_C, _P), 0)
        sel = jnp.where(csub == conf_t.astype(jnp.int32), cf, 0.0)
        ones_c = jnp.ones((1, _C), dtype=jnp.float32)
        ssum = jax.lax.dot_general(
            ones_c, ex, (((1,), (0,)), ((), ())),
            preferred_element_type=jnp.float32)
        xsel = jax.lax.dot_general(
            ones_c, sel, (((1,), (0,)), ((), ())),
            preferred_element_type=jnp.float32)
        ce = jnp.maximum(jnp.log(ssum) + m - xsel, 0.0)

        nl_ref[pl.ds(b, 1), :] = jnp.where(pos, 0.0, ce)
        cepos = jnp.sum(jnp.where(pos, ce, 0.0))
        lane128 = lax.broadcasted_iota(jnp.int32, (1, 128), 1)
        cp_ref[pl.ds(b, 1), :] = jnp.where(lane128 == 0, cepos, 0.0)

    @pl.when(b == _B)
    def _finalize():
        meta = meta_ref[:, 0, :]
        npos = meta[:, 0:1]
        k = jnp.minimum(npos.astype(jnp.int32) * _NEG_POS, _P - 1)
        v = nl_ref[...]
        vi = lax.bitcast_convert_type(v, jnp.int32)

        hi0 = jnp.max(vi, axis=1, keepdims=True)
        lo0 = jnp.full_like(hi0, -1)

        def cnt_ge(mid):
            c = jnp.sum(vi > mid, axis=1, keepdims=True, dtype=jnp.int32)
            return c >= k

        def step(_, carry):
            lo, hi = carry
            d = hi - lo
            h = d // 2
            m1 = lo + d // 4
            m2 = lo + h
            m3 = m2 + (d - h) // 2
            p1 = cnt_ge(m1)
            p2 = cnt_ge(m2)
            p3 = cnt_ge(m3)
            lo = jnp.where(p3, m3, jnp.where(p2, m2, jnp.where(p1, m1, lo)))
            hi = jnp.where(~p1, m1, jnp.where(~p2, m2, jnp.where(~p3, m3, hi)))
            return lo, hi

        _, t_i = lax.fori_loop(0, 16, step, (lo0, hi0))
        t_f = lax.bitcast_convert_type(t_i, jnp.float32)
        gt = vi > t_i
        cnt_t = jnp.sum(gt, axis=1, keepdims=True, dtype=jnp.int32)
        sum_gt = jnp.sum(jnp.where(gt, v, 0.0), axis=1, keepdims=True)
        topk = sum_gt + (k - cnt_t).astype(jnp.float32) * t_f

        n = jnp.sum(npos)
        ll = jnp.sum(meta[:, 1:2])
        lc = jnp.sum(cp_ref[:, 0:1]) + jnp.sum(topk)
        subi = lax.broadcasted_iota(jnp.int32, (8, 128), 0)
        out_ref[...] = jnp.where(subi == 0, ll / n,
                                 jnp.where(subi == 1, lc / n, 0.0))


def kernel(loc_data, conf_data, priors, targets):
    loc_t = loc_data.transpose(0, 2, 1)
    conf_t = conf_data.transpose(0, 2, 1)
    pri_t = priors.T

    ct, meta = pl.pallas_call(
        _match_body,
        grid=(_B,),
        in_specs=[
            pl.BlockSpec((1, 4, _P), lambda b: (b, 0, 0)),
            pl.BlockSpec((4, _P), lambda b: (0, 0)),
            pl.BlockSpec((1, _O, 5), lambda b: (b, 0, 0)),
            pl.BlockSpec((1, 5, _O), lambda b: (b, 0, 0)),
        ],
        out_specs=[
            pl.BlockSpec((1, 1, _P), lambda b: (b, 0, 0)),
            pl.BlockSpec((1, 1, 128), lambda b: (b, 0, 0)),
        ],
        out_shape=[
            jax.ShapeDtypeStruct((_B, 1, _P), jnp.float32),
            jax.ShapeDtypeStruct((_B, 1, 128), jnp.float32),
        ],
    )(loc_t, pri_t, targets, targets.transpose(0, 2, 1))

    out = pl.pallas_call(
        _loss_body,
        grid=(_B + 1,),
        in_specs=[
            pl.BlockSpec((1, _C, _P), lambda b: (jnp.minimum(b, _B - 1), 0, 0)),
            pl.BlockSpec((1, 1, _P), lambda b: (jnp.minimum(b, _B - 1), 0, 0)),
            pl.BlockSpec((_B, 1, 128), lambda b: (0, 0, 0)),
        ],
        out_specs=pl.BlockSpec((8, 128), lambda b: (0, 0)),
        out_shape=jax.ShapeDtypeStruct((8, 128), jnp.float32),
        scratch_shapes=[
            pltpu.VMEM((_B, _P), jnp.float32),
            pltpu.VMEM((_B, 128), jnp.float32),
        ],
    )(conf_t, ct, meta)

    return out[0, 0], out[1, 0]
